# baseline (speedup 1.0000x reference)
import functools

import jax
import jax.numpy as jnp
from jax import lax
from jax.experimental import pallas as pl
from jax.experimental.pallas import tpu as pltpu
from jax.experimental.pallas import tpu_sc as plsc

N = 10000
E = 320000
IN_DIM = 128
HID0 = 32
FD1 = 144

LANES = 128
EROWS = 2560
EPAD = EROWS * LANES
NW = 32
RANGE = 320
NPAD = NW * RANGE
CH = 8192
CHR = CH // LANES
NCH = EPAD // CH
LCAP = EPAD + NCH * LANES

_SC_PARAMS = pltpu.CompilerParams(use_tc_tiling_on_sc=False,
                                  needs_layout_passes=False)


def _mesh():
    return plsc.VectorSubcoreMesh(core_axis_name="c", subcore_axis_name="s")


def _scalar(vec, l):
    return jnp.squeeze(lax.slice(vec, (l,), (l + 1,)))


@functools.lru_cache(maxsize=None)
def _make_prep():
    @functools.partial(
        pl.kernel,
        out_type=(jax.ShapeDtypeStruct((NW, LCAP), jnp.int32),
                  jax.ShapeDtypeStruct((NW, LCAP), jnp.int32),
                  jax.ShapeDtypeStruct((NW, 16), jnp.int32)),
        mesh=_mesh(),
        scratch_types=[
            pltpu.VMEM((CHR, LANES), jnp.int32),
            pltpu.VMEM((CHR, LANES), jnp.int32),
            pltpu.VMEM((CH + LANES,), jnp.int32),
            pltpu.VMEM((CH + LANES,), jnp.int32),
            pltpu.VMEM((48,), jnp.int32),
            pltpu.VMEM((16,), jnp.int32),
        ],
        compiler_params=_SC_PARAMS,
    )
    def prep(srcr_hbm, dstr_hbm, sl_hbm, dl_hbm, cnt_hbm,
             sbuf, dbuf, cs, cd, tmp, cbuf):
        c = lax.axis_index("c")
        s = lax.axis_index("s")
        w = c * 16 + s
        lo = w * RANGE
        tmp[pl.ds(0, 16)] = jnp.zeros((16,), jnp.int32)

        def chunk_body(ch, gnb):
            pltpu.sync_copy(srcr_hbm.at[pl.ds(ch * CHR, CHR)], sbuf)
            pltpu.sync_copy(dstr_hbm.at[pl.ds(ch * CHR, CHR)], dbuf)

            def scan_row(r, cc):
                for k in range(LANES // 16):
                    sv = sbuf[r, pl.ds(k * 16, 16)]
                    dv = dbuf[r, pl.ds(k * 16, 16)]
                    dl = dv - lo
                    m = (dl >= 0) & (dl < RANGE)
                    x = m.astype(jnp.int32)
                    for d in (1, 2, 4, 8):
                        tmp[pl.ds(16, 16)] = x
                        x = x + tmp[pl.ds(16 - d, 16)]
                    offs = cc + x - 1
                    plsc.store_scatter(cs, [offs], sv, mask=m)
                    plsc.store_scatter(cd, [offs], dl, mask=m)
                    cc = cc + _scalar(x, 15)
                return cc

            cc = lax.fori_loop(0, CHR, scan_row, jnp.int32(0))
            for k in range(LANES // 16):
                cs[pl.ds(cc + k * 16, 16)] = jnp.zeros((16,), jnp.int32)
                cd[pl.ds(cc + k * 16, 16)] = jnp.full((16,), RANGE, jnp.int32)
            nbc = (cc + LANES - 1) // LANES

            def flush(b, g):
                pltpu.sync_copy(cs.at[pl.ds(b * LANES, LANES)],
                                sl_hbm.at[w].at[pl.ds((g + b) * LANES, LANES)])
                pltpu.sync_copy(cd.at[pl.ds(b * LANES, LANES)],
                                dl_hbm.at[w].at[pl.ds((g + b) * LANES, LANES)])
                return g

            lax.fori_loop(0, nbc, flush, gnb)
            return gnb + nbc

        gnb = lax.fori_loop(0, NCH, chunk_body, jnp.int32(0))
        cbuf[pl.ds(0, 16)] = jnp.full((16,), 1, jnp.int32) * gnb
        pltpu.sync_copy(cbuf, cnt_hbm.at[w])

    return prep


@functools.lru_cache(maxsize=None)
def _make_segsum(fd):
    nk = fd // 16

    @functools.partial(
        pl.kernel,
        out_type=jax.ShapeDtypeStruct((NPAD, fd), jnp.float32),
        mesh=_mesh(),
        scratch_types=[
            pltpu.VMEM((16,), jnp.int32),
            pltpu.VMEM((LANES,), jnp.int32),
            pltpu.VMEM((LANES,), jnp.int32),
            pltpu.VMEM((LANES, fd), jnp.float32),
            pltpu.VMEM((RANGE + 1, fd), jnp.float32),
            pltpu.SemaphoreType.DMA,
        ],
        compiler_params=_SC_PARAMS,
    )
    def segsum(p_hbm, sl_hbm, dl_hbm, cnt_hbm, out_hbm,
               cbuf, sb, db, rows, acc, sem):
        c = lax.axis_index("c")
        s = lax.axis_index("s")
        w = c * 16 + s
        lo = w * RANGE
        zero16 = jnp.zeros((16,), jnp.float32)

        def zrow(r, carry):
            for k in range(nk):
                acc[r, pl.ds(k * 16, 16)] = zero16
            return carry

        lax.fori_loop(0, RANGE + 1, zrow, 0)
        pltpu.sync_copy(cnt_hbm.at[w], cbuf)
        nbt = _scalar(cbuf[pl.ds(0, 16)], 0)

        def batch(b, carry):
            pltpu.sync_copy(sl_hbm.at[w].at[pl.ds(b * LANES, LANES)], sb)
            pltpu.sync_copy(dl_hbm.at[w].at[pl.ds(b * LANES, LANES)], db)
            pltpu.async_copy(p_hbm.at[sb], rows, sem).wait()

            def grp(g, c2):
                dvec = db[pl.ds(g * 16, 16)]
                for l in range(16):
                    row = _scalar(dvec, l)
                    j = g * 16 + l
                    for k in range(nk):
                        acc[row, pl.ds(k * 16, 16)] = (
                            acc[row, pl.ds(k * 16, 16)]
                            + rows[j, pl.ds(k * 16, 16)]
                        )
                return c2

            lax.fori_loop(0, LANES // 16, grp, 0)
            return carry

        lax.fori_loop(0, nbt, batch, 0)
        pltpu.sync_copy(acc.at[pl.ds(0, RANGE)], out_hbm.at[pl.ds(lo, RANGE)])

    return segsum


def _tc_first(x):
    def body(x_ref, p_ref):
        xb = x_ref[...]
        colid = lax.broadcasted_iota(jnp.int32, (N, FD1 - IN_DIM), 1)
        extra = jnp.where(colid == 0, 1.0, 0.0)
        p_ref[...] = jnp.concatenate([xb, extra], axis=1)

    return pl.pallas_call(
        body, out_shape=jax.ShapeDtypeStruct((N, FD1), jnp.float32))(x)


def _tc_dot(agg, w, b):
    def body(agg_ref, w_ref, b_ref, z_ref):
        z_ref[...] = jnp.dot(agg_ref[...], w_ref[...],
                             preferred_element_type=jnp.float32) + b_ref[...]

    return pl.pallas_call(
        body, out_shape=jax.ShapeDtypeStruct((N, w.shape[1]), jnp.float32),
    )(agg, w, b)


def _tc_final(hg, mlp_w, mlp_b):
    def body(hg_ref, w5_ref, b5_ref, w6_ref, b6_ref, w7_ref, b7_ref, out_ref):
        def vdot(v, w):
            t = v.reshape(-1, 1) * w
            while t.shape[0] > 8:
                half = t.shape[0] // 2
                t = t[:half] + t[half:]
            return jnp.sum(t, axis=0, keepdims=True)

        hg = jnp.maximum(vdot(hg_ref[...], w5_ref[...]) + b5_ref[...], 0.0)
        hg = jnp.maximum(vdot(hg, w6_ref[...]) + b6_ref[...], 0.0)
        out_ref[...] = vdot(hg, w7_ref[...]) + b7_ref[...]

    return pl.pallas_call(
        body,
        out_shape=jax.ShapeDtypeStruct((1, 1), jnp.float32),
    )(hg, mlp_w[0], mlp_b[0], mlp_w[1], mlp_b[1], mlp_w[2], mlp_b[2])


def _prep(srcr, dstr):
    return _make_prep()(srcr, dstr)


def _segsum(fd):
    return _make_segsum(fd)


def kernel(x, edge_index, conv_w, conv_b, bn_g, bn_b, mlp_w, mlp_b):
    src = edge_index[0]
    dst = edge_index[1]
    pad = EPAD - E
    srcr = jnp.concatenate([src, jnp.zeros((pad,), jnp.int32)]).reshape(EROWS, LANES)
    dstr = jnp.concatenate([dst, jnp.full((pad,), N, jnp.int32)]).reshape(EROWS, LANES)
    b = [v.reshape(1, -1) for v in conv_b]
    g = [v.reshape(1, -1) for v in bn_g]
    be = [v.reshape(1, -1) for v in bn_b]
    mb = [v.reshape(1, -1) for v in mlp_b]

    sl, dl, cnt = _prep(srcr, dstr)
    xa = _tc_first(x)
    def bn_relu(z, gg, bb):
        m = jnp.mean(z, axis=0)
        v = jnp.var(z, axis=0)
        return jax.nn.relu((z - m) / jnp.sqrt(v + 1e-5) * gg + bb)

    part1 = _segsum(FD1)(xa, sl, dl, cnt)
    degc = jnp.clip(part1[:N, IN_DIM:IN_DIM + 1], 1.0, None)
    h1 = bn_relu(_tc_dot(part1[:N, :IN_DIM] / degc, conv_w[0], b[0]),
                 bn_g[0], bn_b[0])
    part2 = _segsum(HID0)(h1, sl, dl, cnt)
    h2 = bn_relu(_tc_dot(part2[:N, :] / degc, conv_w[1], b[1]),
                 bn_g[1], bn_b[1])
    part3 = _segsum(HID0)(h2, sl, dl, cnt)
    h3 = bn_relu(_tc_dot(part3[:N, :] / degc, conv_w[2], b[2]),
                 bn_g[2], bn_b[2])
    part4 = _segsum(HID0)(h3, sl, dl, cnt)
    h4 = bn_relu(_tc_dot(part4[:N, :] / degc, conv_w[3], b[3]),
                 bn_g[3], bn_b[3])
    hg = jnp.mean(jnp.concatenate([x, h1, h2, h3, h4], axis=1),
                  axis=0, keepdims=True)
    out = _tc_final(hg, mlp_w, mb)
    return out.reshape(-1)


# speedup vs baseline: 9.6149x; 9.6149x over previous
import functools

import jax
import jax.numpy as jnp
from jax import lax
from jax.experimental import pallas as pl
from jax.experimental.pallas import tpu as pltpu
from jax.experimental.pallas import tpu_sc as plsc

N = 10000
E = 320000
IN_DIM = 128
HID0 = 32

LANES = 128
EROWS = 2560
EPAD = EROWS * LANES
NW = 32
RPW = EROWS // NW
NPAD = 10112
ZR = NPAD // 16


@functools.lru_cache(maxsize=None)
def _make_segsum(fd):
    mesh = plsc.VectorSubcoreMesh(core_axis_name="c", subcore_axis_name="s")

    @functools.partial(
        pl.kernel,
        out_type=jax.ShapeDtypeStruct((2, NPAD, fd), jnp.float32),
        mesh=mesh,
        scratch_types=[
            pltpu.VMEM((RPW, LANES), jnp.int32),
            pltpu.VMEM((RPW, LANES), jnp.int32),
            pltpu.VMEM((LANES, fd), jnp.float32),
            pltpu.VMEM_SHARED((NPAD, fd), jnp.float32),
            pltpu.SemaphoreType.DMA,
        ],
        compiler_params=pltpu.CompilerParams(use_tc_tiling_on_sc=False),
    )
    def segsum(p_hbm, srcr_hbm, dstr_hbm, zeros_hbm, out_hbm,
               idx_s, idx_d, rows, acc, sem):
        c = lax.axis_index("c")
        s = lax.axis_index("s")
        w = c * 16 + s
        pltpu.sync_copy(zeros_hbm.at[pl.ds(s * ZR, ZR)], acc.at[pl.ds(s * ZR, ZR)])
        base = w * RPW
        pltpu.sync_copy(srcr_hbm.at[pl.ds(base, RPW)], idx_s)
        pltpu.sync_copy(dstr_hbm.at[pl.ds(base, RPW)], idx_d)
        plsc.subcore_barrier()

        def body(r, carry):
            pltpu.async_copy(p_hbm.at[idx_s.at[r]], rows, sem).wait()
            pltpu.sync_copy(rows, acc.at[idx_d.at[r]], add=True)
            return carry

        lax.fori_loop(0, RPW, body, 0)
        plsc.subcore_barrier()
        pltpu.sync_copy(acc.at[pl.ds(s * ZR, ZR)],
                        out_hbm.at[c].at[pl.ds(s * ZR, ZR)])

    return segsum


def _tc_first(x, w1):
    def body(x_ref, w_ref, p_ref, mx_ref):
        xb = x_ref[...]
        p = jnp.dot(xb, w_ref[...], preferred_element_type=jnp.float32,
                    precision=lax.Precision.HIGHEST)
        colid = lax.broadcasted_iota(jnp.int32, (N, 8), 1)
        extra = jnp.where(colid == 0, 1.0, 0.0)
        p_ref[...] = jnp.concatenate([p, extra], axis=1)
        mx_ref[...] = jnp.mean(xb, axis=0, keepdims=True)

    return pl.pallas_call(
        body,
        out_shape=(jax.ShapeDtypeStruct((N, 40), jnp.float32),
                   jax.ShapeDtypeStruct((1, IN_DIM), jnp.float32)),
    )(x, w1)


def _bn_relu(z, g, be):
    m = jnp.mean(z, axis=0, keepdims=True)
    zc = z - m
    v = jnp.mean(zc * zc, axis=0, keepdims=True)
    return jnp.maximum(zc / jnp.sqrt(v + 1e-5) * g + be, 0.0)


def _tc_layer1(part, b, g, be, wnext):
    def body(part_ref, b_ref, g_ref, be_ref, w_ref, p_ref, invd_ref, mh_ref):
        sacc = part_ref[0, :N, :] + part_ref[1, :N, :]
        invd = 1.0 / jnp.maximum(sacc[:, 32:33], 1.0)
        z = sacc[:, :32] * invd + b_ref[...]
        h = _bn_relu(z, g_ref[...], be_ref[...])
        p_ref[...] = jnp.dot(h, w_ref[...], preferred_element_type=jnp.float32,
                             precision=lax.Precision.HIGHEST)
        invd_ref[...] = invd
        mh_ref[...] = jnp.mean(h, axis=0, keepdims=True)

    return pl.pallas_call(
        body,
        out_shape=(jax.ShapeDtypeStruct((N, HID0), jnp.float32),
                   jax.ShapeDtypeStruct((N, 1), jnp.float32),
                   jax.ShapeDtypeStruct((1, HID0), jnp.float32)),
    )(part, b, g, be, wnext)


def _tc_mid(part, invd, b, g, be, wnext):
    def body(part_ref, invd_ref, b_ref, g_ref, be_ref, *rest):
        if wnext is None:
            p_ref, mh_ref = rest
        else:
            w_ref, p_ref, mh_ref = rest
        sacc = part_ref[0, :N, :] + part_ref[1, :N, :]
        z = sacc * invd_ref[...] + b_ref[...]
        h = _bn_relu(z, g_ref[...], be_ref[...])
        if wnext is None:
            p_ref[...] = h
        else:
            p_ref[...] = jnp.dot(h, w_ref[...],
                                 preferred_element_type=jnp.float32,
                                 precision=lax.Precision.HIGHEST)
        mh_ref[...] = jnp.mean(h, axis=0, keepdims=True)

    out_shape = (jax.ShapeDtypeStruct((N, HID0), jnp.float32),
                 jax.ShapeDtypeStruct((1, HID0), jnp.float32))
    args = (part, invd, b, g, be) if wnext is None else (part, invd, b, g, be, wnext)
    return pl.pallas_call(body, out_shape=out_shape)(*args)


def _tc_final(part, invd, w4, b4, g4, be4, means, mlp_w, mlp_b):
    def body(part_ref, invd_ref, w_ref, b_ref, g_ref, be_ref,
             mx_ref, m1_ref, m2_ref, m3_ref,
             w5_ref, b5_ref, w6_ref, b6_ref, w7_ref, b7_ref, out_ref):
        sacc = part_ref[0, :N, :] + part_ref[1, :N, :]
        a = sacc * invd_ref[...]
        z = jnp.dot(a, w_ref[...], preferred_element_type=jnp.float32,
                    precision=lax.Precision.HIGHEST) + b_ref[...]
        h = _bn_relu(z, g_ref[...], be_ref[...])
        mh4 = jnp.mean(h, axis=0, keepdims=True)
        hg = jnp.concatenate(
            [mx_ref[...], m1_ref[...], m2_ref[...], m3_ref[...], mh4], axis=1)

        def vdot(v, w):
            t = v.reshape(-1, 1) * w
            while t.shape[0] > 8:
                half = t.shape[0] // 2
                t = t[:half] + t[half:]
            return jnp.sum(t, axis=0, keepdims=True)

        hg = jnp.maximum(vdot(hg, w5_ref[...]) + b5_ref[...], 0.0)
        hg = jnp.maximum(vdot(hg, w6_ref[...]) + b6_ref[...], 0.0)
        out_ref[...] = vdot(hg, w7_ref[...]) + b7_ref[...]

    return pl.pallas_call(
        body,
        out_shape=jax.ShapeDtypeStruct((1, 1), jnp.float32),
    )(part, invd, w4, b4, g4, be4, *means, mlp_w[0], mlp_b[0],
      mlp_w[1], mlp_b[1], mlp_w[2], mlp_b[2])


def kernel(x, edge_index, conv_w, conv_b, bn_g, bn_b, mlp_w, mlp_b):
    src = edge_index[0]
    dst = edge_index[1]
    pad = EPAD - E
    srcr = jnp.concatenate([src, jnp.zeros((pad,), jnp.int32)]).reshape(EROWS, LANES)
    dstr = jnp.concatenate([dst, jnp.full((pad,), N, jnp.int32)]).reshape(EROWS, LANES)
    zeros40 = jnp.zeros((NPAD, 40), jnp.float32)
    zeros32 = jnp.zeros((NPAD, 32), jnp.float32)
    b = [v.reshape(1, -1) for v in conv_b]
    g = [v.reshape(1, -1) for v in bn_g]
    be = [v.reshape(1, -1) for v in bn_b]
    mb = [v.reshape(1, -1) for v in mlp_b]

    p1, mx = _tc_first(x, conv_w[0])
    part1 = _make_segsum(40)(p1, srcr, dstr, zeros40)
    p2, invd, mh1 = _tc_layer1(part1, b[0], g[0], be[0], conv_w[1])
    part2 = _make_segsum(32)(p2, srcr, dstr, zeros32)
    p3, mh2 = _tc_mid(part2, invd, b[1], g[1], be[1], conv_w[2])
    part3 = _make_segsum(32)(p3, srcr, dstr, zeros32)
    h3, mh3 = _tc_mid(part3, invd, b[2], g[2], be[2], None)
    part4 = _make_segsum(32)(h3, srcr, dstr, zeros32)
    out = _tc_final(part4, invd, conv_w[3], b[3], g[3], be[3],
                    (mx, mh1, mh2, mh3), mlp_w, mb)
    return out.reshape(-1)
